# baseline (speedup 1.0000x reference)
import functools

import jax
import jax.numpy as jnp
from jax import lax
from jax.experimental import pallas as pl
from jax.experimental.pallas import tpu as pltpu
from jax.experimental.pallas import tpu_sc as plsc


def _transform_body(table_ref, w_ref, b_ref, gamma_ref, beta_ref, out_ref):
    x = table_ref[...]
    h = lax.dot_general(
        x, w_ref[...], (((1,), (1,)), ((), ())),
        preferred_element_type=jnp.float32,
    )
    h = h + b_ref[...]
    mean = jnp.mean(h, axis=1, keepdims=True)
    hc = h - mean
    var = jnp.mean(hc * hc, axis=1, keepdims=True)
    h = hc * lax.rsqrt(var + 1e-5) * gamma_ref[...] + beta_ref[...]
    norm2 = jnp.sum(h * h, axis=1, keepdims=True)
    out_ref[...] = (h * lax.rsqrt(jnp.maximum(norm2, 1e-24))).astype(jnp.bfloat16)


def _transform_table(table, W, b, gamma, beta):
    n, d = table.shape
    return pl.pallas_call(
        _transform_body,
        out_shape=jax.ShapeDtypeStruct((n, d), jnp.bfloat16),
    )(table, W, b.reshape(1, d), gamma.reshape(1, d), beta.reshape(1, d))


def _make_gather(b_total, d):
    info = plsc.get_sparse_core_info()
    nw = info.num_cores * info.num_subcores
    b_per_w = b_total // nw
    chunk = 64
    n_chunks = b_per_w // chunk
    groups = d // 32
    mesh = plsc.VectorSubcoreMesh(core_axis_name="c", subcore_axis_name="s")

    @functools.partial(
        pl.kernel,
        out_type=jax.ShapeDtypeStruct((b_total, d), jnp.int32),
        mesh=mesh,
        scratch_types=[
            pltpu.VMEM((b_per_w,), jnp.int32),
            pltpu.VMEM((chunk, d // 2), jnp.int32),
            pltpu.VMEM((chunk, d // 2), jnp.int32),
            pltpu.VMEM((chunk, d), jnp.int32),
            pltpu.VMEM((chunk, d), jnp.int32),
            pltpu.SemaphoreType.DMA,
            pltpu.SemaphoreType.DMA,
            pltpu.SemaphoreType.DMA,
            pltpu.SemaphoreType.DMA,
        ],
    )
    def gather_k(tab_hbm, idx_hbm, out_hbm, idx_v, bf0, bf1, f0, f1,
                 gsem0, gsem1, ssem0, ssem1):
        wid = lax.axis_index("s") * info.num_cores + lax.axis_index("c")
        base = wid * b_per_w
        bfb = (bf0, bf1)
        fb = (f0, f1)
        gs = (gsem0, gsem1)
        ss = (ssem0, ssem1)
        pltpu.sync_copy(idx_hbm.at[pl.ds(base, b_per_w)], idx_v)
        def convert(bfu, outref):
            def row_body(r, carry):
                for k in range(groups):
                    w = bfu[r, pl.ds(k * 16, 16)]
                    outref[r, pl.ds(k * 32, 16)] = lax.shift_left(w, 16)
                    outref[r, pl.ds(k * 32 + 16, 16)] = w & (-65536)
                return carry
            lax.fori_loop(0, chunk, row_body, 0)

        gat = [None, None]
        sto = [None, None]
        for c in range(min(2, n_chunks)):
            gat[c] = pltpu.async_copy(
                tab_hbm.at[idx_v.at[pl.ds(c * chunk, chunk)]], bfb[c], gs[c])
        for c in range(n_chunks):
            i = c % 2
            gat[i].wait()
            if sto[i] is not None:
                sto[i].wait()
            convert(bfb[i], fb[i])
            sto[i] = pltpu.async_copy(
                fb[i], out_hbm.at[pl.ds(base + c * chunk, chunk)], ss[i])
            if c + 2 < n_chunks:
                gat[i] = pltpu.async_copy(
                    tab_hbm.at[idx_v.at[pl.ds((c + 2) * chunk, chunk)]],
                    bfb[i], gs[i])
        for s in sto:
            if s is not None:
                s.wait()

    return gather_k


def kernel(ids, table, W, b, gamma, beta):
    y_bf = _transform_table(table, W, b, gamma, beta)
    n, d = y_bf.shape
    y_pairs = y_bf.reshape(n, d // 32, 2, 16).transpose(0, 1, 3, 2)
    y_i32 = lax.bitcast_convert_type(y_pairs, jnp.int32).reshape(n, d // 2)
    gather_k = _make_gather(ids.shape[0], d)
    out_i32 = gather_k(y_i32, ids.astype(jnp.int32))
    return lax.bitcast_convert_type(out_i32, jnp.float32)


# speedup vs baseline: 1.7749x; 1.7749x over previous
import functools

import jax
import jax.numpy as jnp
from jax import lax
from jax.experimental import pallas as pl
from jax.experimental.pallas import tpu as pltpu
from jax.experimental.pallas import tpu_sc as plsc


def _transform_body(table_ref, w_ref, b_ref, gamma_ref, beta_ref, out_ref):
    x = table_ref[...]
    h = lax.dot_general(
        x, w_ref[...], (((1,), (1,)), ((), ())),
        preferred_element_type=jnp.float32,
    )
    h = h + b_ref[...]
    mean = jnp.mean(h, axis=1, keepdims=True)
    hc = h - mean
    var = jnp.mean(hc * hc, axis=1, keepdims=True)
    h = hc * lax.rsqrt(var + 1e-5) * gamma_ref[...] + beta_ref[...]
    norm2 = jnp.sum(h * h, axis=1, keepdims=True)
    out_ref[...] = h * lax.rsqrt(jnp.maximum(norm2, 1e-24))


def _transform_table(table, W, b, gamma, beta):
    n, d = table.shape
    blk = 200
    return pl.pallas_call(
        _transform_body,
        grid=(n // blk,),
        in_specs=[
            pl.BlockSpec((blk, d), lambda i: (i, 0)),
            pl.BlockSpec((d, d), lambda i: (0, 0)),
            pl.BlockSpec((1, d), lambda i: (0, 0)),
            pl.BlockSpec((1, d), lambda i: (0, 0)),
            pl.BlockSpec((1, d), lambda i: (0, 0)),
        ],
        out_specs=pl.BlockSpec((blk, d), lambda i: (i, 0)),
        out_shape=jax.ShapeDtypeStruct((n, d), jnp.float32),
    )(table, W, b.reshape(1, d), gamma.reshape(1, d), beta.reshape(1, d))


def _make_gather(b_total, d):
    info = plsc.get_sparse_core_info()
    nw = info.num_cores * info.num_subcores
    b_per_w = b_total // nw
    chunk = 64
    depth = 3
    n_chunks = b_per_w // chunk
    mesh = plsc.VectorSubcoreMesh(core_axis_name="c", subcore_axis_name="s")

    @functools.partial(
        pl.kernel,
        out_type=jax.ShapeDtypeStruct((b_total, d), jnp.float32),
        mesh=mesh,
        scratch_types=(
            [pltpu.VMEM((b_per_w,), jnp.int32)]
            + [pltpu.VMEM((chunk, d), jnp.float32)] * depth
            + [pltpu.SemaphoreType.DMA] * (2 * depth)
        ),
    )
    def gather_k(tab_hbm, idx_hbm, out_hbm, idx_v, *bufs_sems):
        bufs = bufs_sems[:depth]
        gsems = bufs_sems[depth:2 * depth]
        ssems = bufs_sems[2 * depth:]
        wid = lax.axis_index("s") * info.num_cores + lax.axis_index("c")
        base = wid * b_per_w
        pltpu.sync_copy(idx_hbm.at[pl.ds(base, b_per_w)], idx_v)
        gat = [None] * depth
        sto = [None] * depth

        def start_gather(c):
            i = c % depth
            if sto[i] is not None:
                sto[i].wait()
            gat[i] = pltpu.async_copy(
                tab_hbm.at[idx_v.at[pl.ds(c * chunk, chunk)]],
                bufs[i], gsems[i])

        for c in range(min(depth - 1, n_chunks)):
            start_gather(c)
        for c in range(n_chunks):
            i = c % depth
            gat[i].wait()
            sto[i] = pltpu.async_copy(
                bufs[i], out_hbm.at[pl.ds(base + c * chunk, chunk)], ssems[i])
            if c + depth - 1 < n_chunks:
                start_gather(c + depth - 1)
        for s in sto:
            if s is not None:
                s.wait()

    return gather_k


def kernel(ids, table, W, b, gamma, beta):
    y_table = _transform_table(table, W, b, gamma, beta)
    gather_k = _make_gather(ids.shape[0], table.shape[1])
    return gather_k(y_table, ids.astype(jnp.int32))


# speedup vs baseline: 1.8434x; 1.0386x over previous
import functools

import jax
import jax.numpy as jnp
from jax import lax
from jax.experimental import pallas as pl
from jax.experimental.pallas import tpu as pltpu
from jax.experimental.pallas import tpu_sc as plsc


def _transform_body(table_ref, w_ref, b_ref, gamma_ref, beta_ref, out_ref):
    x = table_ref[...]
    h = lax.dot_general(
        x, w_ref[...], (((1,), (1,)), ((), ())),
        preferred_element_type=jnp.float32,
    )
    h = h + b_ref[...]
    mean = jnp.mean(h, axis=1, keepdims=True)
    hc = h - mean
    var = jnp.mean(hc * hc, axis=1, keepdims=True)
    h = hc * lax.rsqrt(var + 1e-5) * gamma_ref[...] + beta_ref[...]
    norm2 = jnp.sum(h * h, axis=1, keepdims=True)
    out_ref[...] = h * lax.rsqrt(jnp.maximum(norm2, 1e-24))


def _transform_table(table, W, b, gamma, beta):
    n, d = table.shape
    return pl.pallas_call(
        _transform_body,
        out_shape=jax.ShapeDtypeStruct((n, d), jnp.float32),
    )(table, W, b.reshape(1, d), gamma.reshape(1, d), beta.reshape(1, d))


def _make_gather(b_total, d):
    info = plsc.get_sparse_core_info()
    nw = info.num_cores * info.num_subcores
    b_per_w = b_total // nw
    chunk = 64
    depth = 3
    n_chunks = b_per_w // chunk
    mesh = plsc.VectorSubcoreMesh(core_axis_name="c", subcore_axis_name="s")

    @functools.partial(
        pl.kernel,
        out_type=jax.ShapeDtypeStruct((b_total, d), jnp.float32),
        mesh=mesh,
        scratch_types=(
            [pltpu.VMEM((b_per_w,), jnp.int32)]
            + [pltpu.VMEM((chunk, d), jnp.float32)] * depth
            + [pltpu.SemaphoreType.DMA] * (2 * depth)
        ),
    )
    def gather_k(tab_hbm, idx_hbm, out_hbm, idx_v, *bufs_sems):
        bufs = bufs_sems[:depth]
        gsems = bufs_sems[depth:2 * depth]
        ssems = bufs_sems[2 * depth:]
        wid = lax.axis_index("s") * info.num_cores + lax.axis_index("c")
        base = wid * b_per_w
        pltpu.sync_copy(idx_hbm.at[pl.ds(base, b_per_w)], idx_v)
        gat = [None] * depth
        sto = [None] * depth

        def start_gather(c):
            i = c % depth
            if sto[i] is not None:
                sto[i].wait()
            gat[i] = pltpu.async_copy(
                tab_hbm.at[idx_v.at[pl.ds(c * chunk, chunk)]],
                bufs[i], gsems[i])

        for c in range(min(depth - 1, n_chunks)):
            start_gather(c)
        for c in range(n_chunks):
            i = c % depth
            gat[i].wait()
            sto[i] = pltpu.async_copy(
                bufs[i], out_hbm.at[pl.ds(base + c * chunk, chunk)], ssems[i])
            if c + depth - 1 < n_chunks:
                start_gather(c + depth - 1)
        for s in sto:
            if s is not None:
                s.wait()

    return gather_k


def kernel(ids, table, W, b, gamma, beta):
    y_table = _transform_table(table, W, b, gamma, beta)
    gather_k = _make_gather(ids.shape[0], table.shape[1])
    return gather_k(y_table, ids.astype(jnp.int32))
